# Initial kernel scaffold; baseline (speedup 1.0000x reference)
import jax
import jax.numpy as jnp
from jax.experimental import pallas as pl


def kernel(state_seq, state_seq_length, token_seq, token_seq_length):
    raise NotImplementedError("write your pallas kernel here")


# speedup vs baseline: 34.7644x; 34.7644x over previous
import functools
import math

import jax
import jax.numpy as jnp
from jax import lax
from jax.experimental import pallas as pl
from jax.experimental.pallas import tpu as pltpu

MAXLEN = 2048
D = 512
C = 256
DC = D + C
S = 1024


def _ts_body(o_ref):
    nt = C // 2
    log_inc = math.log(10000.0) / (nt - 1.0)
    j = lax.broadcasted_iota(jnp.int32, (MAXLEN, nt), 1).astype(jnp.float32)
    p = lax.broadcasted_iota(jnp.int32, (MAXLEN, nt), 0).astype(jnp.float32)
    st = p * jnp.exp(j * (-log_inc))
    o_ref[:, 0:nt] = jnp.sin(st)
    o_ref[:, nt:C] = jnp.cos(st)


def _main_body(sl_ref, tl_ref, state_ref, token_ref, ts_ref, out_ref, len_ref):
    b = pl.program_id(0)
    sl = sl_ref[b]
    tl = tl_ref[b]
    ln = jnp.minimum(sl + tl, MAXLEN)
    len_ref[b] = ln

    rows = lax.broadcasted_iota(jnp.int32, (S, D), 0)
    tokm = jnp.where(rows < tl, token_ref[0], 0.0)
    tokpad = jnp.concatenate([tokm, jnp.zeros((S, D), jnp.float32)], axis=0)
    rolled = pltpu.roll(tokpad, sl, 0)
    r2d = lax.broadcasted_iota(jnp.int32, (MAXLEN, D), 0)
    statepad = jnp.concatenate(
        [state_ref[0], jnp.zeros((S, D), jnp.float32)], axis=0)
    out_ref[0, :, 0:D] = jnp.where(r2d < sl, statepad, rolled)

    r2 = lax.broadcasted_iota(jnp.int32, (MAXLEN, C), 0)
    out_ref[0, :, D:DC] = jnp.where(r2 < ln, ts_ref[...], 0.0)


@jax.jit
def kernel(state_seq, state_seq_length, token_seq, token_seq_length):
    B = state_seq.shape[0]
    ts = pl.pallas_call(
        _ts_body,
        out_shape=jax.ShapeDtypeStruct((MAXLEN, C), jnp.float32),
    )()
    out, ln = pl.pallas_call(
        _main_body,
        grid=(B,),
        in_specs=[
            pl.BlockSpec(memory_space=pltpu.SMEM),
            pl.BlockSpec(memory_space=pltpu.SMEM),
            pl.BlockSpec((1, S, D), lambda b: (b, 0, 0)),
            pl.BlockSpec((1, S, D), lambda b: (b, 0, 0)),
            pl.BlockSpec((MAXLEN, C), lambda b: (0, 0)),
        ],
        out_specs=[
            pl.BlockSpec((1, MAXLEN, DC), lambda b: (b, 0, 0)),
            pl.BlockSpec(memory_space=pltpu.SMEM),
        ],
        out_shape=[
            jax.ShapeDtypeStruct((B, MAXLEN, DC), jnp.float32),
            jax.ShapeDtypeStruct((B,), jnp.int32),
        ],
    )(state_seq_length.astype(jnp.int32), token_seq_length.astype(jnp.int32),
      state_seq, token_seq, ts)
    return out, ln
